# Initial kernel scaffold; baseline (speedup 1.0000x reference)
import jax
import jax.numpy as jnp
from jax.experimental import pallas as pl


def kernel(nf_user, nf_item, emb_user_w, emb_user_b, emb_item_w, emb_item_b, att0_w_self, att0_b_self, att0_w_lin, att0_b_lin, att0_w_att, att0_b_att, att0_w_res, att0_b_res, att1_w_self, att1_b_self, att1_w_lin, att1_b_lin, att1_w_att, att1_b_att, att1_w_res, att1_b_res, att2_w_self, att2_b_self, att2_w_lin, att2_b_lin, att2_w_att, att2_b_att, att2_w_res, att2_b_res, ei0, ew0, ei1, ew1, ei2, ew2):
    raise NotImplementedError("write your pallas kernel here")


# speedup vs baseline: 2.2318x; 2.2318x over previous
import functools

import jax
import jax.numpy as jnp
from jax import lax
from jax.experimental import pallas as pl
from jax.experimental.pallas import tpu as pltpu

_mm = functools.partial(lax.dot_general, preferred_element_type=jnp.float32)


def _emb_kernel(nfu_ref, nfi_ref, wu_ref, wi_ref, bu_ref, bi_ref, o_ref):
    xu = _mm(nfu_ref[...].astype(jnp.bfloat16), wu_ref[...],
             (((1,), (0,)), ((), ()))) + bu_ref[...]
    xi = _mm(nfi_ref[...].astype(jnp.bfloat16), wi_ref[...],
             (((1,), (0,)), ((), ()))) + bi_ref[...]
    o_ref[0] = xu.astype(jnp.bfloat16)
    o_ref[1] = xi.astype(jnp.bfloat16)


def _embed(nf_user, nf_item, wu_t, wi_t, bu, bi, *, blocks=8):
    n, in_f = nf_user.shape
    of = wu_t.shape[1]
    blk = n // blocks
    out = pl.pallas_call(
        _emb_kernel,
        out_shape=jax.ShapeDtypeStruct((2, n, of), jnp.bfloat16),
        grid=(blocks,),
        in_specs=[
            pl.BlockSpec((blk, in_f), lambda i: (i, 0)),
            pl.BlockSpec((blk, in_f), lambda i: (i, 0)),
            pl.BlockSpec((in_f, of), lambda i: (0, 0)),
            pl.BlockSpec((in_f, of), lambda i: (0, 0)),
            pl.BlockSpec((1, of), lambda i: (0, 0)),
            pl.BlockSpec((1, of), lambda i: (0, 0)),
        ],
        out_specs=pl.BlockSpec((2, blk, of), lambda i: (0, i, 0)),
        compiler_params=pltpu.CompilerParams(
            dimension_semantics=("parallel",)),
    )(nf_user, nf_item, wu_t, wi_t, bu, bi)
    return out.reshape(2 * n, of)


def _rel_kernel(x_ref, wlr_ref, blr_ref, wself_ref, bself_ref,
                wcat_ref, batt_ref, src_ref, tgt_ref, ew_ref, o_ref,
                *, num_heads, fdim):
    of = num_heads * fdim
    n = x_ref.shape[0]
    e = src_ref.shape[1]

    x = x_ref[...]
    z = _mm(x, wlr_ref[...], (((1,), (0,)), ((), ()))) + blr_ref[...]
    h32 = z[:, :of]
    res = z[:, of:]
    h = h32.astype(jnp.bfloat16)

    hs = (_mm(h, wself_ref[...], (((1,), (0,)), ((), ())))
          + bself_ref[...]).astype(jnp.bfloat16)

    hcat = jnp.concatenate([h, hs], axis=1)
    p = _mm(hcat, wcat_ref[...], (((1,), (0,)), ((), ())))
    pb = p.astype(jnp.bfloat16)

    ids = lax.broadcasted_iota(jnp.int32, (n, e), 0)
    one = jnp.float32(1.0)
    zero = jnp.float32(0.0)
    sn = jnp.where(ids == src_ref[...], one, zero).astype(jnp.bfloat16)
    tn = jnp.where(ids == tgt_ref[...], one, zero).astype(jnp.bfloat16)

    a = _mm(sn, pb, (((0,), (0,)), ((), ())))
    b = _mm(tn, pb, (((0,), (0,)), ((), ())))
    logit = a[:, :num_heads] + b[:, num_heads:] + batt_ref[...]
    logit = jnp.where(logit >= 0, logit, jnp.float32(0.2) * logit)

    gmax = jnp.max(logit, axis=0, keepdims=True)
    ex = jnp.exp(logit - gmax)
    dnode = _mm(sn, ex.astype(jnp.bfloat16),
                (((1,), (0,)), ((), ())))
    den = _mm(sn, dnode.astype(jnp.bfloat16),
              (((0,), (0,)), ((), ())))
    alpha = ex / jnp.maximum(den, jnp.float32(1e-20))

    we = (ew_ref[...] * alpha).astype(jnp.bfloat16)
    rowh = lax.broadcasted_iota(jnp.int32, (num_heads, of), 0)
    colh = lax.broadcasted_iota(jnp.int32, (num_heads, of), 1) // fdim
    expand = jnp.where(rowh == colh, one, zero).astype(jnp.bfloat16)
    wfull = _mm(we, expand, (((1,), (0,)), ((), ())))
    xsrc = _mm(sn, h, (((0,), (0,)), ((), ())))
    msg = (wfull * xsrc).astype(jnp.bfloat16)
    agg = _mm(tn, msg, (((1,), (0,)), ((), ())))
    o_ref[...] = agg + res


def _relations(x, wlr_t, blr, wself_t, bself, wcat, batt, src, tgt, ew_t,
               *, num_heads, fdim):
    n, of = x.shape
    r = wlr_t.shape[0]
    e = src.shape[2]
    kfn = functools.partial(_rel_kernel, num_heads=num_heads, fdim=fdim)
    per_rel = lambda i: (i, 0, 0)
    out = pl.pallas_call(
        kfn,
        out_shape=jax.ShapeDtypeStruct((n, r * of), jnp.float32),
        grid=(r,),
        in_specs=[
            pl.BlockSpec((n, of), lambda i: (0, 0)),
            pl.BlockSpec((None, of, 2 * of), per_rel),
            pl.BlockSpec((None, 1, 2 * of), per_rel),
            pl.BlockSpec((None, of, of), per_rel),
            pl.BlockSpec((None, 1, of), per_rel),
            pl.BlockSpec((None, 2 * of, 2 * num_heads), per_rel),
            pl.BlockSpec((None, 1, 1), per_rel),
            pl.BlockSpec((None, 1, e), per_rel),
            pl.BlockSpec((None, 1, e), per_rel),
            pl.BlockSpec((None, e, 1), per_rel),
        ],
        out_specs=pl.BlockSpec((n, of), lambda i: (0, i)),
        compiler_params=pltpu.CompilerParams(
            dimension_semantics=("parallel",)),
    )(x, wlr_t, blr, wself_t, bself, wcat, batt, src, tgt, ew_t)
    return out


def kernel(nf_user, nf_item, emb_user_w, emb_user_b, emb_item_w, emb_item_b,
           att0_w_self, att0_b_self, att0_w_lin, att0_b_lin,
           att0_w_att, att0_b_att, att0_w_res, att0_b_res,
           att1_w_self, att1_b_self, att1_w_lin, att1_b_lin,
           att1_w_att, att1_b_att, att1_w_res, att1_b_res,
           att2_w_self, att2_b_self, att2_w_lin, att2_b_lin,
           att2_w_att, att2_b_att, att2_w_res, att2_b_res,
           ei0, ew0, ei1, ew1, ei2, ew2):
    num_heads = 8
    of = emb_user_w.shape[0]
    fdim = of // num_heads
    bf16 = jnp.bfloat16

    wu_t = emb_user_w.T.astype(bf16)
    wi_t = emb_item_w.T.astype(bf16)
    bu = emb_user_b.reshape(1, of)
    bi = emb_item_b.reshape(1, of)

    eye_h = jnp.eye(num_heads, dtype=jnp.float32)
    wlr_l, blr_l, wself_l, bself_l, wcat_l, batt_l = [], [], [], [], [], []
    for w_self, b_self, w_lin, b_lin, w_att, b_att, w_res, b_res in (
            (att0_w_self, att0_b_self, att0_w_lin, att0_b_lin,
             att0_w_att, att0_b_att, att0_w_res, att0_b_res),
            (att1_w_self, att1_b_self, att1_w_lin, att1_b_lin,
             att1_w_att, att1_b_att, att1_w_res, att1_b_res),
            (att2_w_self, att2_b_self, att2_w_lin, att2_b_lin,
             att2_w_att, att2_b_att, att2_w_res, att2_b_res)):
        wlr_l.append(jnp.concatenate([w_lin, w_res], axis=0).T.astype(bf16))
        blr_l.append(jnp.concatenate([b_lin, b_res])[None, :])
        wself_l.append(jnp.kron(eye_h, w_self).T.astype(bf16))
        bself_l.append(jnp.tile(b_self, num_heads)[None, :])
        wa = w_att.reshape(3 * fdim)
        wsrc = jnp.kron(eye_h, wa[:fdim].reshape(fdim, 1))
        wtgt = jnp.kron(eye_h, wa[fdim:2 * fdim].reshape(fdim, 1))
        wsa = jnp.kron(eye_h, wa[2 * fdim:].reshape(fdim, 1))
        top = jnp.concatenate([wsrc, wtgt], axis=1)
        bot = jnp.concatenate([wsa, jnp.zeros_like(wtgt)], axis=1)
        wcat_l.append(jnp.concatenate([top, bot], axis=0).astype(bf16))
        batt_l.append(b_att.reshape(1, 1))
    wlr_t = jnp.stack(wlr_l)
    blr = jnp.stack(blr_l)
    wself_t = jnp.stack(wself_l)
    bself = jnp.stack(bself_l)
    wcat = jnp.stack(wcat_l)
    batt = jnp.stack(batt_l)

    src = jnp.stack([ei0[0][None, :], ei1[0][None, :], ei2[0][None, :]])
    tgt = jnp.stack([ei0[1][None, :], ei1[1][None, :], ei2[1][None, :]])
    ew_t = jnp.stack([ew0[:, None], ew1[:, None], ew2[:, None]])

    x = _embed(nf_user, nf_item, wu_t, wi_t, bu, bi)

    out = _relations(x, wlr_t, blr, wself_t, bself, wcat, batt,
                     src, tgt, ew_t, num_heads=num_heads, fdim=fdim)
    n = x.shape[0]
    r = wlr_t.shape[0]
    return out.reshape(n * r, of)


# speedup vs baseline: 2.6678x; 1.1954x over previous
import functools

import jax
import jax.numpy as jnp
from jax import lax
from jax.experimental import pallas as pl
from jax.experimental.pallas import tpu as pltpu

_mm = functools.partial(lax.dot_general, preferred_element_type=jnp.float32)


def _emb_kernel(nfu_ref, nfi_ref, wu_ref, wi_ref, bu_ref, bi_ref, o_ref):
    xu = _mm(nfu_ref[...].astype(jnp.bfloat16), wu_ref[...],
             (((1,), (0,)), ((), ()))) + bu_ref[...]
    xi = _mm(nfi_ref[...].astype(jnp.bfloat16), wi_ref[...],
             (((1,), (0,)), ((), ()))) + bi_ref[...]
    o_ref[0] = xu.astype(jnp.bfloat16)
    o_ref[1] = xi.astype(jnp.bfloat16)


def _embed(nf_user, nf_item, wu_t, wi_t, bu, bi, *, blocks=8):
    n, in_f = nf_user.shape
    of = wu_t.shape[1]
    blk = n // blocks
    out = pl.pallas_call(
        _emb_kernel,
        out_shape=jax.ShapeDtypeStruct((2, n, of), jnp.bfloat16),
        grid=(blocks,),
        in_specs=[
            pl.BlockSpec((blk, in_f), lambda i: (i, 0)),
            pl.BlockSpec((blk, in_f), lambda i: (i, 0)),
            pl.BlockSpec((in_f, of), lambda i: (0, 0)),
            pl.BlockSpec((in_f, of), lambda i: (0, 0)),
            pl.BlockSpec((1, of), lambda i: (0, 0)),
            pl.BlockSpec((1, of), lambda i: (0, 0)),
        ],
        out_specs=pl.BlockSpec((2, blk, of), lambda i: (0, i, 0)),
        compiler_params=pltpu.CompilerParams(
            dimension_semantics=("parallel",)),
    )(nf_user, nf_item, wu_t, wi_t, bu, bi)
    return out.reshape(2 * n, of)


def _rel_kernel(x_ref, wlr_ref, blr_ref, wself_ref, bself_ref,
                wcat_ref, batt_ref, srcc_ref, srcr_ref, tgtc_ref, tgtr_ref,
                ew_ref, o_ref, *, num_heads, fdim):
    of = num_heads * fdim
    n = x_ref.shape[0]
    e = srcr_ref.shape[1]
    one = jnp.float32(1.0)
    zero = jnp.float32(0.0)
    bf16 = jnp.bfloat16

    x = x_ref[...]
    z = _mm(x, wlr_ref[...], (((1,), (0,)), ((), ()))) + blr_ref[...]
    h32 = z[:, :of]
    res = z[:, of:]
    h = h32.astype(bf16)

    hs = (_mm(h, wself_ref[...], (((1,), (0,)), ((), ())))
          + bself_ref[...]).astype(bf16)

    hcat = jnp.concatenate([h, hs], axis=1)
    p = _mm(hcat, wcat_ref[...], (((1,), (0,)), ((), ())))
    pb = p.astype(bf16)

    ids_en = lax.broadcasted_iota(jnp.int32, (e, n), 1)
    se = jnp.where(ids_en == srcc_ref[...], one, zero).astype(bf16)
    te = jnp.where(ids_en == tgtc_ref[...], one, zero).astype(bf16)
    ids_ne = lax.broadcasted_iota(jnp.int32, (n, e), 0)
    tn = jnp.where(ids_ne == tgtr_ref[...], one, zero).astype(bf16)
    grp = jnp.where(srcc_ref[...] == srcr_ref[...], one, zero).astype(bf16)

    hp = jnp.concatenate([h, pb], axis=1)
    g = _mm(se, hp, (((1,), (0,)), ((), ())))
    b2 = _mm(te, pb, (((1,), (0,)), ((), ())))
    xsrc = g[:, :of]

    logit = (g[:, of:of + num_heads] + b2[:, num_heads:]
             + batt_ref[...])
    logit = jnp.where(logit >= 0, logit, jnp.float32(0.2) * logit)

    gmax = jnp.max(logit, axis=0, keepdims=True)
    ex = jnp.exp(logit - gmax)
    den = _mm(grp, ex.astype(bf16), (((1,), (0,)), ((), ())))
    alpha = ex / jnp.maximum(den, jnp.float32(1e-20))

    we = (ew_ref[...] * alpha).astype(bf16)
    rowh = lax.broadcasted_iota(jnp.int32, (num_heads, of), 0)
    colh = lax.broadcasted_iota(jnp.int32, (num_heads, of), 1) // fdim
    expand = jnp.where(rowh == colh, one, zero).astype(bf16)
    wfull = _mm(we, expand, (((1,), (0,)), ((), ())))
    msg = (wfull * xsrc).astype(bf16)
    agg = _mm(tn, msg, (((1,), (0,)), ((), ())))
    o_ref[...] = agg + res


def _relations(x, wlr_t, blr, wself_t, bself, wcat, batt,
               src_c, src_r, tgt_c, tgt_r, ew_t, *, num_heads, fdim):
    n, of = x.shape
    r = wlr_t.shape[0]
    e = src_r.shape[2]
    kfn = functools.partial(_rel_kernel, num_heads=num_heads, fdim=fdim)
    per_rel = lambda i: (i, 0, 0)
    out = pl.pallas_call(
        kfn,
        out_shape=jax.ShapeDtypeStruct((n, r * of), jnp.float32),
        grid=(r,),
        in_specs=[
            pl.BlockSpec((n, of), lambda i: (0, 0)),
            pl.BlockSpec((None, of, 2 * of), per_rel),
            pl.BlockSpec((None, 1, 2 * of), per_rel),
            pl.BlockSpec((None, of, of), per_rel),
            pl.BlockSpec((None, 1, of), per_rel),
            pl.BlockSpec((None, 2 * of, 2 * num_heads), per_rel),
            pl.BlockSpec((None, 1, 1), per_rel),
            pl.BlockSpec((None, e, 1), per_rel),
            pl.BlockSpec((None, 1, e), per_rel),
            pl.BlockSpec((None, e, 1), per_rel),
            pl.BlockSpec((None, 1, e), per_rel),
            pl.BlockSpec((None, e, 1), per_rel),
        ],
        out_specs=pl.BlockSpec((n, of), lambda i: (0, i)),
        compiler_params=pltpu.CompilerParams(
            dimension_semantics=("parallel",)),
    )(x, wlr_t, blr, wself_t, bself, wcat, batt,
      src_c, src_r, tgt_c, tgt_r, ew_t)
    return out


def kernel(nf_user, nf_item, emb_user_w, emb_user_b, emb_item_w, emb_item_b,
           att0_w_self, att0_b_self, att0_w_lin, att0_b_lin,
           att0_w_att, att0_b_att, att0_w_res, att0_b_res,
           att1_w_self, att1_b_self, att1_w_lin, att1_b_lin,
           att1_w_att, att1_b_att, att1_w_res, att1_b_res,
           att2_w_self, att2_b_self, att2_w_lin, att2_b_lin,
           att2_w_att, att2_b_att, att2_w_res, att2_b_res,
           ei0, ew0, ei1, ew1, ei2, ew2):
    num_heads = 8
    of = emb_user_w.shape[0]
    fdim = of // num_heads
    bf16 = jnp.bfloat16

    wu_t = emb_user_w.T.astype(bf16)
    wi_t = emb_item_w.T.astype(bf16)
    bu = emb_user_b.reshape(1, of)
    bi = emb_item_b.reshape(1, of)

    eye_h = jnp.eye(num_heads, dtype=jnp.float32)
    wlr_l, blr_l, wself_l, bself_l, wcat_l, batt_l = [], [], [], [], [], []
    for w_self, b_self, w_lin, b_lin, w_att, b_att, w_res, b_res in (
            (att0_w_self, att0_b_self, att0_w_lin, att0_b_lin,
             att0_w_att, att0_b_att, att0_w_res, att0_b_res),
            (att1_w_self, att1_b_self, att1_w_lin, att1_b_lin,
             att1_w_att, att1_b_att, att1_w_res, att1_b_res),
            (att2_w_self, att2_b_self, att2_w_lin, att2_b_lin,
             att2_w_att, att2_b_att, att2_w_res, att2_b_res)):
        wlr_l.append(jnp.concatenate([w_lin, w_res], axis=0).T.astype(bf16))
        blr_l.append(jnp.concatenate([b_lin, b_res])[None, :])
        wself_l.append(jnp.kron(eye_h, w_self).T.astype(bf16))
        bself_l.append(jnp.tile(b_self, num_heads)[None, :])
        wa = w_att.reshape(3 * fdim)
        wsrc = jnp.kron(eye_h, wa[:fdim].reshape(fdim, 1))
        wtgt = jnp.kron(eye_h, wa[fdim:2 * fdim].reshape(fdim, 1))
        wsa = jnp.kron(eye_h, wa[2 * fdim:].reshape(fdim, 1))
        top = jnp.concatenate([wsrc, wtgt], axis=1)
        bot = jnp.concatenate([wsa, jnp.zeros_like(wtgt)], axis=1)
        wcat_l.append(jnp.concatenate([top, bot], axis=0).astype(bf16))
        batt_l.append(b_att.reshape(1, 1))
    wlr_t = jnp.stack(wlr_l)
    blr = jnp.stack(blr_l)
    wself_t = jnp.stack(wself_l)
    bself = jnp.stack(bself_l)
    wcat = jnp.stack(wcat_l)
    batt = jnp.stack(batt_l)

    src_c = jnp.stack([ei0[0][:, None], ei1[0][:, None], ei2[0][:, None]])
    src_r = jnp.stack([ei0[0][None, :], ei1[0][None, :], ei2[0][None, :]])
    tgt_c = jnp.stack([ei0[1][:, None], ei1[1][:, None], ei2[1][:, None]])
    tgt_r = jnp.stack([ei0[1][None, :], ei1[1][None, :], ei2[1][None, :]])
    ew_t = jnp.stack([ew0[:, None], ew1[:, None], ew2[:, None]])

    x = _embed(nf_user, nf_item, wu_t, wi_t, bu, bi)

    out = _relations(x, wlr_t, blr, wself_t, bself, wcat, batt,
                     src_c, src_r, tgt_c, tgt_r, ew_t,
                     num_heads=num_heads, fdim=fdim)
    n = x.shape[0]
    r = wlr_t.shape[0]
    return out.reshape(n * r, of)


# speedup vs baseline: 4.3285x; 1.6225x over previous
import functools

import jax
import jax.numpy as jnp
from jax import lax
from jax.experimental import pallas as pl
from jax.experimental.pallas import tpu as pltpu

_mm = functools.partial(lax.dot_general, preferred_element_type=jnp.float32)
_DN = (((1,), (0,)), ((), ()))
_DT = (((1,), (1,)), ((), ()))


def _emb_kernel(nfu_ref, nfi_ref, wu_ref, wi_ref, bu_ref, bi_ref, o_ref):
    wu = wu_ref[...].astype(jnp.bfloat16)
    wi = wi_ref[...].astype(jnp.bfloat16)
    xu = _mm(nfu_ref[...].astype(jnp.bfloat16), wu, _DT) + bu_ref[...]
    xi = _mm(nfi_ref[...].astype(jnp.bfloat16), wi, _DT) + bi_ref[...]
    o_ref[0] = xu.astype(jnp.bfloat16)
    o_ref[1] = xi.astype(jnp.bfloat16)


def _embed(nf_user, nf_item, w_u, w_i, bu, bi, *, blocks=8):
    n, in_f = nf_user.shape
    of = w_u.shape[0]
    blk = n // blocks
    full = lambda i: (0, 0)
    out = pl.pallas_call(
        _emb_kernel,
        out_shape=jax.ShapeDtypeStruct((2, n, of), jnp.bfloat16),
        grid=(blocks,),
        in_specs=[
            pl.BlockSpec((blk, in_f), lambda i: (i, 0)),
            pl.BlockSpec((blk, in_f), lambda i: (i, 0)),
            pl.BlockSpec((of, in_f), full),
            pl.BlockSpec((of, in_f), full),
            pl.BlockSpec((1, of), full),
            pl.BlockSpec((1, of), full),
        ],
        out_specs=pl.BlockSpec((2, blk, of), lambda i: (0, i, 0)),
        compiler_params=pltpu.CompilerParams(
            dimension_semantics=("parallel",)),
    )(nf_user, nf_item, w_u, w_i, bu, bi)
    return out.reshape(2 * n, of)


def _sel3(r, a0, a1, a2):
    return jnp.where(r == 0, a0, jnp.where(r == 1, a1, a2))


def _rel_kernel(x_ref,
                ws0_ref, bs0_ref, wl0_ref, bl0_ref, wa0_ref, ba0_ref,
                wr0_ref, br0_ref, ei0_ref, ew0_ref,
                ws1_ref, bs1_ref, wl1_ref, bl1_ref, wa1_ref, ba1_ref,
                wr1_ref, br1_ref, ei1_ref, ew1_ref,
                ws2_ref, bs2_ref, wl2_ref, bl2_ref, wa2_ref, ba2_ref,
                wr2_ref, br2_ref, ei2_ref, ew2_ref,
                o_ref, *, num_heads, fdim):
    of = num_heads * fdim
    n = x_ref.shape[0]
    e = ei0_ref.shape[1]
    one = jnp.float32(1.0)
    zero = jnp.float32(0.0)
    bf16 = jnp.bfloat16
    r = pl.program_id(0)

    ws = _sel3(r, ws0_ref[...], ws1_ref[...], ws2_ref[...])
    bs = _sel3(r, bs0_ref[...], bs1_ref[...], bs2_ref[...])
    wl = _sel3(r, wl0_ref[...], wl1_ref[...], wl2_ref[...])
    bl = _sel3(r, bl0_ref[...], bl1_ref[...], bl2_ref[...])
    wa = _sel3(r, wa0_ref[...], wa1_ref[...], wa2_ref[...])
    ba = _sel3(r, ba0_ref[...], ba1_ref[...], ba2_ref[...])
    wr = _sel3(r, wr0_ref[...], wr1_ref[...], wr2_ref[...])
    br = _sel3(r, br0_ref[...], br1_ref[...], br2_ref[...])
    ei = _sel3(r, ei0_ref[...], ei1_ref[...], ei2_ref[...])
    ew = _sel3(r, ew0_ref[...], ew1_ref[...], ew2_ref[...])

    wlr = jnp.concatenate([wl, wr], axis=0).astype(bf16)
    blr = jnp.concatenate([bl, br], axis=1)
    blki = lax.broadcasted_iota(jnp.int32, (of, of), 0) // fdim
    blkj = lax.broadcasted_iota(jnp.int32, (of, of), 1) // fdim
    wst = jnp.concatenate([ws] * num_heads, axis=1)
    wst = jnp.concatenate([wst] * num_heads, axis=0)
    wself = jnp.where(blki == blkj, wst, zero).astype(bf16)
    bst = jnp.concatenate([bs] * num_heads, axis=1)
    wat = jnp.transpose(wa, (1, 0))
    m8 = (lax.broadcasted_iota(jnp.int32, (of, num_heads), 0) // fdim
          == lax.broadcasted_iota(jnp.int32, (of, num_heads), 1))
    wsrc = jnp.where(m8, jnp.concatenate([wat[:fdim]] * num_heads, 0), zero)
    wtgt = jnp.where(m8, jnp.concatenate([wat[fdim:2 * fdim]] * num_heads, 0),
                     zero)
    wsa = jnp.where(m8, jnp.concatenate([wat[2 * fdim:]] * num_heads, 0),
                    zero)
    top = jnp.concatenate([wsrc, wtgt], axis=1)
    bot = jnp.concatenate([wsa, jnp.zeros_like(wtgt)], axis=1)
    wcat = jnp.concatenate([top, bot], axis=0).astype(bf16)

    srcr = ei[0:1, :]
    tgtr = ei[1:2, :]
    srcc = jnp.transpose(srcr, (1, 0))
    tgtc = jnp.transpose(tgtr, (1, 0))

    x = x_ref[...]
    z = _mm(x, wlr, _DT) + blr
    h32 = z[:, :of]
    res = z[:, of:]
    h = h32.astype(bf16)

    hs = (_mm(h, wself, _DT) + bst).astype(bf16)

    hcat = jnp.concatenate([h, hs], axis=1)
    p = _mm(hcat, wcat, _DN)
    pb = p.astype(bf16)

    ids_en = lax.broadcasted_iota(jnp.int32, (e, n), 1)
    se = jnp.where(ids_en == srcc, one, zero).astype(bf16)
    te = jnp.where(ids_en == tgtc, one, zero).astype(bf16)
    ids_ne = lax.broadcasted_iota(jnp.int32, (n, e), 0)
    tn = jnp.where(ids_ne == tgtr, one, zero).astype(bf16)
    grp = jnp.where(srcc == srcr, one, zero).astype(bf16)

    hp = jnp.concatenate([h, pb], axis=1)
    g = _mm(se, hp, _DN)
    b2 = _mm(te, pb, _DN)
    xsrc = g[:, :of]

    logit = g[:, of:of + num_heads] + b2[:, num_heads:] + ba
    logit = jnp.where(logit >= 0, logit, jnp.float32(0.2) * logit)

    gmax = jnp.max(logit, axis=0, keepdims=True)
    ex = jnp.exp(logit - gmax)
    den = _mm(grp, ex.astype(bf16), _DN)
    alpha = ex / jnp.maximum(den, jnp.float32(1e-20))

    we = (ew * alpha).astype(bf16)
    rowh = lax.broadcasted_iota(jnp.int32, (num_heads, of), 0)
    colh = lax.broadcasted_iota(jnp.int32, (num_heads, of), 1) // fdim
    expand = jnp.where(rowh == colh, one, zero).astype(bf16)
    wfull = _mm(we, expand, _DN)
    msg = (wfull * xsrc).astype(bf16)
    agg = _mm(tn, msg, _DN)
    o_ref[...] = agg + res


def _relations(x, rel_args, *, num_heads, fdim):
    n, of = x.shape
    r = 3
    e = rel_args[8].shape[1]
    fdim_ = fdim
    kfn = functools.partial(_rel_kernel, num_heads=num_heads, fdim=fdim_)
    full = lambda i: (0, 0)
    per_rel_specs = [
        pl.BlockSpec((fdim, fdim), full),
        pl.BlockSpec((1, fdim), full),
        pl.BlockSpec((of, of), full),
        pl.BlockSpec((1, of), full),
        pl.BlockSpec((1, 3 * fdim), full),
        pl.BlockSpec((1, 1), full),
        pl.BlockSpec((of, of), full),
        pl.BlockSpec((1, of), full),
        pl.BlockSpec((2, e), full),
        pl.BlockSpec((e, 1), full),
    ]
    out = pl.pallas_call(
        kfn,
        out_shape=jax.ShapeDtypeStruct((n, r * of), jnp.float32),
        grid=(r,),
        in_specs=[pl.BlockSpec((n, of), full)] + per_rel_specs * 3,
        out_specs=pl.BlockSpec((n, of), lambda i: (0, i)),
        compiler_params=pltpu.CompilerParams(
            dimension_semantics=("parallel",)),
    )(x, *rel_args)
    return out


def kernel(nf_user, nf_item, emb_user_w, emb_user_b, emb_item_w, emb_item_b,
           att0_w_self, att0_b_self, att0_w_lin, att0_b_lin,
           att0_w_att, att0_b_att, att0_w_res, att0_b_res,
           att1_w_self, att1_b_self, att1_w_lin, att1_b_lin,
           att1_w_att, att1_b_att, att1_w_res, att1_b_res,
           att2_w_self, att2_b_self, att2_w_lin, att2_b_lin,
           att2_w_att, att2_b_att, att2_w_res, att2_b_res,
           ei0, ew0, ei1, ew1, ei2, ew2):
    num_heads = 8
    of = emb_user_w.shape[0]
    fdim = of // num_heads

    x = _embed(nf_user, nf_item, emb_user_w, emb_item_w,
               emb_user_b[None, :], emb_item_b[None, :])

    rel_args = (
        att0_w_self, att0_b_self[None, :], att0_w_lin, att0_b_lin[None, :],
        att0_w_att, att0_b_att[None, :], att0_w_res, att0_b_res[None, :],
        ei0, ew0[:, None],
        att1_w_self, att1_b_self[None, :], att1_w_lin, att1_b_lin[None, :],
        att1_w_att, att1_b_att[None, :], att1_w_res, att1_b_res[None, :],
        ei1, ew1[:, None],
        att2_w_self, att2_b_self[None, :], att2_w_lin, att2_b_lin[None, :],
        att2_w_att, att2_b_att[None, :], att2_w_res, att2_b_res[None, :],
        ei2, ew2[:, None],
    )
    out = _relations(x, rel_args, num_heads=num_heads, fdim=fdim)
    n = x.shape[0]
    return out.reshape(n * 3, of)
